# baseline (speedup 1.0000x reference)
import functools

import jax
import jax.numpy as jnp
from jax import lax
from jax.experimental import pallas as pl
from jax.experimental.pallas import tpu as pltpu
from jax.experimental.pallas import tpu_sc as plsc

B = 32
C = 512
P = 2048
MAX_H = 70
MAX_W = 40
HW = MAX_H * MAX_W
CHUNK = 56
NCHUNK = HW // CHUNK
CC = 256
FC = 128
L = 16
I32MAX = 2147483647
I32MIN = -2147483648


def _transpose_body(x_ref, o_ref, m_ref):
    x = x_ref[0]
    m_ref[0, 0] = jnp.all(x != -1.0, axis=0).astype(jnp.int32)
    xt = x.T.astype(jnp.bfloat16)
    lo = lax.bitcast_convert_type(xt[:, :C // 2], jnp.uint16)
    hi = lax.bitcast_convert_type(xt[:, C // 2:], jnp.uint16)
    o_ref[0] = jnp.bitwise_or(
        lax.shift_left(hi.astype(jnp.int32), 16), lo.astype(jnp.int32))


def _transpose_feats(features):
    return pl.pallas_call(
        _transpose_body,
        grid=(B,),
        in_specs=[pl.BlockSpec((1, C, P), lambda b: (b, 0, 0))],
        out_specs=[
            pl.BlockSpec((1, P, C // 2), lambda b: (b, 0, 0)),
            pl.BlockSpec((1, 1, P), lambda b: (b, 0, 0)),
        ],
        out_shape=[
            jax.ShapeDtypeStruct((B, P, C // 2), jnp.int32),
            jax.ShapeDtypeStruct((B, 1, P), jnp.int32),
        ],
    )(features)


def _sc_body(ys_hbm, xs_hbm, tab_hbm, rm_hbm, val_hbm, gath_hbm,
             ys_v, xs_v, cell_v, pt_v, ptc_v, rm_v, val_v,
             buf0, buf1, sem0, sem1):
    b = lax.axis_index("c") * 16 + lax.axis_index("s")
    pltpu.sync_copy(ys_hbm.at[b], ys_v)
    pltpu.sync_copy(xs_hbm.at[b], xs_v)
    pltpu.sync_copy(rm_hbm.at[b], rm_v)

    iota = lax.iota(jnp.int32, L)

    def mm_body(i, carry):
        mny, mxy, mnx, mxx = carry
        yv = ys_v[pl.ds(i * L, L)]
        xv = xs_v[pl.ds(i * L, L)]
        return (jnp.minimum(mny, yv), jnp.maximum(mxy, yv),
                jnp.minimum(mnx, xv), jnp.maximum(mxx, xv))

    big = jnp.full((L,), I32MAX, jnp.int32)
    small = jnp.full((L,), I32MIN, jnp.int32)
    mny, mxy, mnx, mxx = lax.fori_loop(
        0, P // L, mm_body, (big, small, big, small))

    def _allreduce(v, op):
        for s in (8, 4, 2, 1):
            ptc_v[pl.ds(0, L)] = v
            g = plsc.load_gather(ptc_v, [jnp.bitwise_and(iota + s, L - 1)])
            v = op(v, g)
        return v

    min_y = _allreduce(mny, jnp.minimum)
    max_y = _allreduce(mxy, jnp.maximum)
    min_x = _allreduce(mnx, jnp.minimum)
    max_x = _allreduce(mxx, jnp.maximum)
    h = max_y - min_y + 1
    w = max_x - min_x + 1
    one = jnp.full((L,), 1, jnp.int32)
    zero = jnp.full((L,), 0, jnp.int32)
    si = jnp.where(w > h, one, zero)
    h2 = si * w + (one - si) * h
    w2 = si * h + (one - si) * w
    ofh = (MAX_H - h2 + 1) // 2
    ofw = (MAX_W - w2 + 1) // 2

    def cell_body(i, _):
        yv = ys_v[pl.ds(i * L, L)] - min_y
        xv = xs_v[pl.ds(i * L, L)] - min_x
        iout = si * xv + (one - si) * yv + ofh
        jout = si * yv + (one - si) * xv + ofw
        cell_v[pl.ds(i * L, L)] = iout * MAX_W + jout
        return 0

    lax.fori_loop(0, P // L, cell_body, 0)

    def init_body(i, _):
        pt_v[pl.ds(i * L, L)] = jnp.full((L,), jnp.int32(-1))
        return 0

    lax.fori_loop(0, HW // L, init_body, 0)

    perms = [jnp.bitwise_and(iota + r, L - 1) for r in range(1, L)]
    vmasks = [iota < (L - r) for r in range(1, L)]

    def dedup_body(i, _):
        base = i * L
        c = cell_v[pl.ds(base, L)]
        dup = iota < 0
        for r in range(1, L):
            g = plsc.load_gather(cell_v, [base + perms[r - 1]])
            dup = jnp.logical_or(
                dup, jnp.logical_and(g == c, vmasks[r - 1]))
        plsc.store_scatter(pt_v, [c], base + iota,
                           mask=jnp.logical_not(dup))
        return 0

    lax.fori_loop(0, P // L, dedup_body, 0)

    boff = b * P

    def clamp_body(i, _):
        v = pt_v[pl.ds(i * L, L)]
        vc = jnp.maximum(v, 0)
        rm = plsc.load_gather(rm_v, [vc])
        ok = jnp.logical_and(v >= 0, rm != 0)
        val_v[pl.ds(i * L, L)] = jnp.where(ok, one, zero)
        ptc_v[pl.ds(i * L, L)] = vc + boff
        return 0

    lax.fori_loop(0, HW // L, clamp_body, 0)

    pltpu.sync_copy(val_v, val_hbm.at[b])

    def _start(g, buf, sem):
        idx = ptc_v.at[pl.ds(g * CHUNK, CHUNK)]
        pltpu.async_copy(tab_hbm.at[idx], buf, sem)

    def _drain(buf, sem):
        pltpu.make_async_copy(tab_hbm.at[pl.ds(0, CHUNK)], buf, sem).wait()

    _start(0, buf0, sem0)

    def gath_body(i, _):
        g0 = i * 2
        g1 = g0 + 1
        _start(g1, buf1, sem1)
        _drain(buf0, sem0)
        pltpu.sync_copy(buf0, gath_hbm.at[b, pl.ds(g0 * CHUNK, CHUNK)])

        @pl.when(g1 + 1 < NCHUNK)
        def _():
            _start(g1 + 1, buf0, sem0)

        _drain(buf1, sem1)
        pltpu.sync_copy(buf1, gath_hbm.at[b, pl.ds(g1 * CHUNK, CHUNK)])
        return 0

    lax.fori_loop(0, NCHUNK // 2, gath_body, 0)


_sc_mesh = plsc.VectorSubcoreMesh(core_axis_name="c", subcore_axis_name="s")

_sc_call = functools.partial(
    pl.kernel,
    out_type=(
        jax.ShapeDtypeStruct((B, HW), jnp.int32),
        jax.ShapeDtypeStruct((B, HW, C // 2), jnp.int32),
    ),
    mesh=_sc_mesh,
    compiler_params=pltpu.CompilerParams(needs_layout_passes=False),
    scratch_types=[
        pltpu.VMEM((P,), jnp.int32),
        pltpu.VMEM((P,), jnp.int32),
        pltpu.VMEM((P,), jnp.int32),
        pltpu.VMEM((HW,), jnp.int32),
        pltpu.VMEM((HW,), jnp.int32),
        pltpu.VMEM((P,), jnp.int32),
        pltpu.VMEM((HW,), jnp.int32),
        pltpu.VMEM((CHUNK, C // 2), jnp.int32),
        pltpu.VMEM((CHUNK, C // 2), jnp.int32),
        pltpu.SemaphoreType.DMA,
        pltpu.SemaphoreType.DMA,
    ],
)(_sc_body)


def _finish_body(eye_ref, g_ref, v_ref, bk_ref, o_ref):
    y = g_ref[0]
    v = v_ref[0, 0] != 0
    lo = lax.bitcast_convert_type(y.astype(jnp.uint16), jnp.bfloat16)
    hi = lax.bitcast_convert_type(
        lax.shift_right_logical(y, 16).astype(jnp.uint16), jnp.bfloat16)
    dn = (((1,), (1,)), ((), ()))
    eye = eye_ref[...]
    lo_t = lax.dot_general(eye, lo, dn,
                           preferred_element_type=jnp.float32)
    hi_t = lax.dot_general(eye, hi, dn,
                           preferred_element_type=jnp.float32)
    xt = jnp.concatenate([lo_t, hi_t], axis=0)
    o_ref[0] = jnp.where(v[None, :], xt, bk_ref[...])


GBF = 8
NGF = B // GBF


def _finish(gath, valid, backend_feature, eye):
    vr = valid.reshape(B, 1, HW)
    bk2 = backend_feature.reshape(C, 1)
    outs = []
    for g in range(NGF):
        g0 = g * GBF
        out_g = pl.pallas_call(
            _finish_body,
            grid=(GBF,),
            in_specs=[
                pl.BlockSpec((C // 2, C // 2), lambda b: (0, 0)),
                pl.BlockSpec((1, HW, C // 2),
                             lambda b, g0=g0: (b + g0, 0, 0)),
                pl.BlockSpec((1, 1, HW), lambda b, g0=g0: (b + g0, 0, 0)),
                pl.BlockSpec((C, 1), lambda b: (0, 0)),
            ],
            out_specs=pl.BlockSpec((1, C, HW), lambda b: (b, 0, 0)),
            out_shape=jax.ShapeDtypeStruct((GBF, C, HW), jnp.float32),
        )(eye, gath, vr, bk2)
        outs.append(out_g.reshape(GBF, C, MAX_H, MAX_W))
    return jnp.concatenate(outs, axis=0)


def kernel(features, ys, xs, validation, backend_feature):
    feats = features.astype(jnp.float32)
    ysi = ys.astype(jnp.int32)
    xsi = xs.astype(jnp.int32)
    featT, rowmask = _transpose_feats(feats)
    tab = featT.reshape(B * P, C // 2)
    valid, gath = _sc_call(ysi, xsi, tab, rowmask.reshape(B, P))
    eye = jnp.eye(C // 2, dtype=jnp.bfloat16)
    return _finish(gath, valid, backend_feature.astype(jnp.float32), eye)


# speedup vs baseline: 1.2203x; 1.2203x over previous
import functools

import jax
import jax.numpy as jnp
from jax import lax
from jax.experimental import pallas as pl
from jax.experimental.pallas import tpu as pltpu
from jax.experimental.pallas import tpu_sc as plsc

B = 32
C = 512
P = 2048
MAX_H = 70
MAX_W = 40
HW = MAX_H * MAX_W
CHUNK = 112
NCHUNK = HW // CHUNK
CC = 256
FC = 128
L = 16
I32MAX = 2147483647
I32MIN = -2147483648


def _transpose_body(x_ref, o_ref, m_ref):
    x = x_ref[0]
    m_ref[0, 0] = jnp.all(x != -1.0, axis=0).astype(jnp.int32)
    xt = x.T.astype(jnp.bfloat16)
    lo = lax.bitcast_convert_type(xt[:, :C // 2], jnp.uint16)
    hi = lax.bitcast_convert_type(xt[:, C // 2:], jnp.uint16)
    o_ref[0] = jnp.bitwise_or(
        lax.shift_left(hi.astype(jnp.int32), 16), lo.astype(jnp.int32))


def _transpose_feats(features):
    return pl.pallas_call(
        _transpose_body,
        grid=(B,),
        in_specs=[pl.BlockSpec((1, C, P), lambda b: (b, 0, 0))],
        out_specs=[
            pl.BlockSpec((1, P, C // 2), lambda b: (b, 0, 0)),
            pl.BlockSpec((1, 1, P), lambda b: (b, 0, 0)),
        ],
        out_shape=[
            jax.ShapeDtypeStruct((B, P, C // 2), jnp.int32),
            jax.ShapeDtypeStruct((B, 1, P), jnp.int32),
        ],
    )(features)


def _sc_body(ys_hbm, xs_hbm, tab_hbm, rm_hbm, val_hbm, gath_hbm,
             ys_v, xs_v, cell_v, pt_v, ptc_v, rm_v, val_v,
             buf0, buf1, sem0, sem1):
    b = lax.axis_index("c") * 16 + lax.axis_index("s")
    pltpu.sync_copy(ys_hbm.at[b], ys_v)
    pltpu.sync_copy(xs_hbm.at[b], xs_v)
    pltpu.sync_copy(rm_hbm.at[b], rm_v)

    iota = lax.iota(jnp.int32, L)

    def mm_body(i, carry):
        mny, mxy, mnx, mxx = carry
        yv = ys_v[pl.ds(i * L, L)]
        xv = xs_v[pl.ds(i * L, L)]
        return (jnp.minimum(mny, yv), jnp.maximum(mxy, yv),
                jnp.minimum(mnx, xv), jnp.maximum(mxx, xv))

    big = jnp.full((L,), I32MAX, jnp.int32)
    small = jnp.full((L,), I32MIN, jnp.int32)
    mny, mxy, mnx, mxx = lax.fori_loop(
        0, P // L, mm_body, (big, small, big, small))

    def _allreduce(v, op):
        for s in (8, 4, 2, 1):
            ptc_v[pl.ds(0, L)] = v
            g = plsc.load_gather(ptc_v, [jnp.bitwise_and(iota + s, L - 1)])
            v = op(v, g)
        return v

    min_y = _allreduce(mny, jnp.minimum)
    max_y = _allreduce(mxy, jnp.maximum)
    min_x = _allreduce(mnx, jnp.minimum)
    max_x = _allreduce(mxx, jnp.maximum)
    h = max_y - min_y + 1
    w = max_x - min_x + 1
    one = jnp.full((L,), 1, jnp.int32)
    zero = jnp.full((L,), 0, jnp.int32)
    si = jnp.where(w > h, one, zero)
    h2 = si * w + (one - si) * h
    w2 = si * h + (one - si) * w
    ofh = (MAX_H - h2 + 1) // 2
    ofw = (MAX_W - w2 + 1) // 2

    def cell_body(i, _):
        yv = ys_v[pl.ds(i * L, L)] - min_y
        xv = xs_v[pl.ds(i * L, L)] - min_x
        iout = si * xv + (one - si) * yv + ofh
        jout = si * yv + (one - si) * xv + ofw
        cell_v[pl.ds(i * L, L)] = iout * MAX_W + jout
        return 0

    lax.fori_loop(0, P // L, cell_body, 0)

    def init_body(i, _):
        pt_v[pl.ds(i * L, L)] = jnp.full((L,), jnp.int32(-1))
        return 0

    lax.fori_loop(0, HW // L, init_body, 0)

    perms = [jnp.bitwise_and(iota + r, L - 1) for r in range(1, L)]
    vmasks = [iota < (L - r) for r in range(1, L)]

    def dedup_body(i, _):
        base = i * L
        c = cell_v[pl.ds(base, L)]
        dup = iota < 0
        for r in range(1, L):
            g = plsc.load_gather(cell_v, [base + perms[r - 1]])
            dup = jnp.logical_or(
                dup, jnp.logical_and(g == c, vmasks[r - 1]))
        plsc.store_scatter(pt_v, [c], base + iota,
                           mask=jnp.logical_not(dup))
        return 0

    lax.fori_loop(0, P // L, dedup_body, 0)

    boff = b * P

    def clamp_body(i, _):
        v = pt_v[pl.ds(i * L, L)]
        vc = jnp.maximum(v, 0)
        rm = plsc.load_gather(rm_v, [vc])
        ok = jnp.logical_and(v >= 0, rm != 0)
        val_v[pl.ds(i * L, L)] = jnp.where(ok, one, zero)
        ptc_v[pl.ds(i * L, L)] = vc + boff
        return 0

    lax.fori_loop(0, HW // L, clamp_body, 0)

    pltpu.sync_copy(val_v, val_hbm.at[b])

    def _start(g, buf, sem):
        idx = ptc_v.at[pl.ds(g * CHUNK, CHUNK)]
        pltpu.async_copy(tab_hbm.at[idx], buf, sem)

    def _drain(buf, sem):
        pltpu.make_async_copy(tab_hbm.at[pl.ds(0, CHUNK)], buf, sem).wait()

    _start(0, buf0, sem0)

    def gath_body(i, _):
        g0 = i * 2
        g1 = g0 + 1
        _start(g1, buf1, sem1)
        _drain(buf0, sem0)
        pltpu.sync_copy(buf0, gath_hbm.at[b, pl.ds(g0 * CHUNK, CHUNK)])

        @pl.when(g1 + 1 < NCHUNK)
        def _():
            _start(g1 + 1, buf0, sem0)

        _drain(buf1, sem1)
        pltpu.sync_copy(buf1, gath_hbm.at[b, pl.ds(g1 * CHUNK, CHUNK)])
        return 0

    lax.fori_loop(0, NCHUNK // 2, gath_body, 0)
    if NCHUNK % 2:
        _drain(buf0, sem0)
        pltpu.sync_copy(
            buf0, gath_hbm.at[b, pl.ds((NCHUNK - 1) * CHUNK, CHUNK)])


_sc_mesh = plsc.VectorSubcoreMesh(core_axis_name="c", subcore_axis_name="s")

_sc_call = functools.partial(
    pl.kernel,
    out_type=(
        jax.ShapeDtypeStruct((B, HW), jnp.int32),
        jax.ShapeDtypeStruct((B, HW, C // 2), jnp.int32),
    ),
    mesh=_sc_mesh,
    compiler_params=pltpu.CompilerParams(needs_layout_passes=False),
    scratch_types=[
        pltpu.VMEM((P,), jnp.int32),
        pltpu.VMEM((P,), jnp.int32),
        pltpu.VMEM((P,), jnp.int32),
        pltpu.VMEM((HW,), jnp.int32),
        pltpu.VMEM((HW,), jnp.int32),
        pltpu.VMEM((P,), jnp.int32),
        pltpu.VMEM((HW,), jnp.int32),
        pltpu.VMEM((CHUNK, C // 2), jnp.int32),
        pltpu.VMEM((CHUNK, C // 2), jnp.int32),
        pltpu.SemaphoreType.DMA,
        pltpu.SemaphoreType.DMA,
    ],
)(_sc_body)


def _finish_body(eye_ref, g_ref, v_ref, bk_ref, o_ref):
    y = g_ref[0]
    v = v_ref[0, 0] != 0
    lo = lax.bitcast_convert_type(y.astype(jnp.uint16), jnp.bfloat16)
    hi = lax.bitcast_convert_type(
        lax.shift_right_logical(y, 16).astype(jnp.uint16), jnp.bfloat16)
    dn = (((1,), (1,)), ((), ()))
    eye = eye_ref[...]
    lo_t = lax.dot_general(eye, lo, dn,
                           preferred_element_type=jnp.float32)
    hi_t = lax.dot_general(eye, hi, dn,
                           preferred_element_type=jnp.float32)
    xt = jnp.concatenate([lo_t, hi_t], axis=0)
    o_ref[0] = jnp.where(v[None, :], xt, bk_ref[...])


def _finish(gath, valid, backend_feature, eye):
    vr = valid.reshape(B, 1, HW)
    bk2 = backend_feature.reshape(C, 1)
    out = pl.pallas_call(
        _finish_body,
        grid=(B,),
        in_specs=[
            pl.BlockSpec((C // 2, C // 2), lambda b: (0, 0)),
            pl.BlockSpec((1, HW, C // 2), lambda b: (b, 0, 0)),
            pl.BlockSpec((1, 1, HW), lambda b: (b, 0, 0)),
            pl.BlockSpec((C, 1), lambda b: (0, 0)),
        ],
        out_specs=pl.BlockSpec((1, C, HW), lambda b: (b, 0, 0)),
        out_shape=jax.ShapeDtypeStruct((B, C, HW), jnp.float32),
    )(eye, gath, vr, bk2)
    return out.reshape(B, C, MAX_H, MAX_W)


def kernel(features, ys, xs, validation, backend_feature):
    feats = features.astype(jnp.float32)
    ysi = ys.astype(jnp.int32)
    xsi = xs.astype(jnp.int32)
    featT, rowmask = _transpose_feats(feats)
    tab = featT.reshape(B * P, C // 2)
    valid, gath = _sc_call(ysi, xsi, tab, rowmask.reshape(B, P))
    eye = jnp.eye(C // 2, dtype=jnp.bfloat16)
    return _finish(gath, valid, backend_feature.astype(jnp.float32), eye)


# speedup vs baseline: 1.2560x; 1.0292x over previous
import functools

import jax
import jax.numpy as jnp
from jax import lax
from jax.experimental import pallas as pl
from jax.experimental.pallas import tpu as pltpu
from jax.experimental.pallas import tpu_sc as plsc

B = 32
C = 512
P = 2048
MAX_H = 70
MAX_W = 40
HW = MAX_H * MAX_W
CHUNK = 56
NCHUNK = HW // CHUNK
CC = 256
FC = 128
L = 16
I32MAX = 2147483647
I32MIN = -2147483648


def _transpose_body(x_ref, o_ref, m_ref):
    x = x_ref[0]
    m_ref[0, 0] = jnp.all(x != -1.0, axis=0).astype(jnp.int32)
    xt = x.T.astype(jnp.bfloat16)
    lo = lax.bitcast_convert_type(xt[:, :C // 2], jnp.uint16)
    hi = lax.bitcast_convert_type(xt[:, C // 2:], jnp.uint16)
    o_ref[0] = jnp.bitwise_or(
        lax.shift_left(hi.astype(jnp.int32), 16), lo.astype(jnp.int32))


def _transpose_feats(features):
    return pl.pallas_call(
        _transpose_body,
        grid=(B,),
        in_specs=[pl.BlockSpec((1, C, P), lambda b: (b, 0, 0))],
        out_specs=[
            pl.BlockSpec((1, P, C // 2), lambda b: (b, 0, 0)),
            pl.BlockSpec((1, 1, P), lambda b: (b, 0, 0)),
        ],
        out_shape=[
            jax.ShapeDtypeStruct((B, P, C // 2), jnp.int32),
            jax.ShapeDtypeStruct((B, 1, P), jnp.int32),
        ],
    )(features)


def _sc_body(ys_hbm, xs_hbm, tab_hbm, rm_hbm, val_hbm, gath_hbm,
             ys_v, xs_v, cell_v, pt_v, ptc_v, rm_v, val_v,
             buf0, buf1, sem0, sem1):
    b = lax.axis_index("c") * 16 + lax.axis_index("s")
    pltpu.sync_copy(ys_hbm.at[b], ys_v)
    pltpu.sync_copy(xs_hbm.at[b], xs_v)
    pltpu.sync_copy(rm_hbm.at[b], rm_v)

    iota = lax.iota(jnp.int32, L)

    def mm_body(i, carry):
        mny, mxy, mnx, mxx = carry
        yv = ys_v[pl.ds(i * L, L)]
        xv = xs_v[pl.ds(i * L, L)]
        return (jnp.minimum(mny, yv), jnp.maximum(mxy, yv),
                jnp.minimum(mnx, xv), jnp.maximum(mxx, xv))

    big = jnp.full((L,), I32MAX, jnp.int32)
    small = jnp.full((L,), I32MIN, jnp.int32)
    mny, mxy, mnx, mxx = lax.fori_loop(
        0, P // L, mm_body, (big, small, big, small))

    def _allreduce(v, op):
        for s in (8, 4, 2, 1):
            ptc_v[pl.ds(0, L)] = v
            g = plsc.load_gather(ptc_v, [jnp.bitwise_and(iota + s, L - 1)])
            v = op(v, g)
        return v

    min_y = _allreduce(mny, jnp.minimum)
    max_y = _allreduce(mxy, jnp.maximum)
    min_x = _allreduce(mnx, jnp.minimum)
    max_x = _allreduce(mxx, jnp.maximum)
    h = max_y - min_y + 1
    w = max_x - min_x + 1
    one = jnp.full((L,), 1, jnp.int32)
    zero = jnp.full((L,), 0, jnp.int32)
    si = jnp.where(w > h, one, zero)
    h2 = si * w + (one - si) * h
    w2 = si * h + (one - si) * w
    ofh = (MAX_H - h2 + 1) // 2
    ofw = (MAX_W - w2 + 1) // 2

    def cell_body(i, _):
        yv = ys_v[pl.ds(i * L, L)] - min_y
        xv = xs_v[pl.ds(i * L, L)] - min_x
        iout = si * xv + (one - si) * yv + ofh
        jout = si * yv + (one - si) * xv + ofw
        cell_v[pl.ds(i * L, L)] = iout * MAX_W + jout
        return 0

    lax.fori_loop(0, P // L, cell_body, 0)

    def init_body(i, _):
        pt_v[pl.ds(i * L, L)] = jnp.full((L,), jnp.int32(-1))
        return 0

    lax.fori_loop(0, HW // L, init_body, 0)

    perms = [jnp.bitwise_and(iota + r, L - 1) for r in range(1, L)]
    vmasks = [iota < (L - r) for r in range(1, L)]

    def dedup_body(i, _):
        base = i * L
        c = cell_v[pl.ds(base, L)]
        dup = iota < 0
        for r in range(1, L):
            g = plsc.load_gather(cell_v, [base + perms[r - 1]])
            dup = jnp.logical_or(
                dup, jnp.logical_and(g == c, vmasks[r - 1]))
        plsc.store_scatter(pt_v, [c], base + iota,
                           mask=jnp.logical_not(dup))
        return 0

    lax.fori_loop(0, P // L, dedup_body, 0)

    boff = b * P

    def clamp_body(i, _):
        v = pt_v[pl.ds(i * L, L)]
        vc = jnp.maximum(v, 0)
        rm = plsc.load_gather(rm_v, [vc])
        ok = jnp.logical_and(v >= 0, rm != 0)
        val_v[pl.ds(i * L, L)] = jnp.where(ok, one, zero)
        ptc_v[pl.ds(i * L, L)] = vc + boff
        return 0

    lax.fori_loop(0, HW // L, clamp_body, 0)

    pltpu.sync_copy(val_v, val_hbm.at[b])

    def _start(g, buf, sem):
        idx = ptc_v.at[pl.ds(g * CHUNK, CHUNK)]
        pltpu.async_copy(tab_hbm.at[idx], buf, sem)

    def _drain(buf, sem):
        pltpu.make_async_copy(tab_hbm.at[pl.ds(0, CHUNK)], buf, sem).wait()

    _start(0, buf0, sem0)

    def gath_body(i, _):
        g0 = i * 2
        g1 = g0 + 1
        _start(g1, buf1, sem1)
        _drain(buf0, sem0)
        pltpu.sync_copy(buf0, gath_hbm.at[b, pl.ds(g0 * CHUNK, CHUNK)])

        @pl.when(g1 + 1 < NCHUNK)
        def _():
            _start(g1 + 1, buf0, sem0)

        _drain(buf1, sem1)
        pltpu.sync_copy(buf1, gath_hbm.at[b, pl.ds(g1 * CHUNK, CHUNK)])
        return 0

    lax.fori_loop(0, NCHUNK // 2, gath_body, 0)


_sc_mesh = plsc.VectorSubcoreMesh(core_axis_name="c", subcore_axis_name="s")

_sc_call = functools.partial(
    pl.kernel,
    out_type=(
        jax.ShapeDtypeStruct((B, HW), jnp.int32),
        jax.ShapeDtypeStruct((B, HW, C // 2), jnp.int32),
    ),
    mesh=_sc_mesh,
    compiler_params=pltpu.CompilerParams(needs_layout_passes=False),
    scratch_types=[
        pltpu.VMEM((P,), jnp.int32),
        pltpu.VMEM((P,), jnp.int32),
        pltpu.VMEM((P,), jnp.int32),
        pltpu.VMEM((HW,), jnp.int32),
        pltpu.VMEM((HW,), jnp.int32),
        pltpu.VMEM((P,), jnp.int32),
        pltpu.VMEM((HW,), jnp.int32),
        pltpu.VMEM((CHUNK, C // 2), jnp.int32),
        pltpu.VMEM((CHUNK, C // 2), jnp.int32),
        pltpu.SemaphoreType.DMA,
        pltpu.SemaphoreType.DMA,
    ],
)(_sc_body)


def _finish_body(eye_ref, g_ref, v_ref, bk_ref, o_ref):
    y = g_ref[0]
    v = v_ref[0, 0] != 0
    lo = lax.bitcast_convert_type(y.astype(jnp.uint16), jnp.bfloat16)
    hi = lax.bitcast_convert_type(
        lax.shift_right_logical(y, 16).astype(jnp.uint16), jnp.bfloat16)
    dn = (((1,), (1,)), ((), ()))
    eye = eye_ref[...]
    lo_t = lax.dot_general(eye, lo, dn,
                           preferred_element_type=jnp.float32)
    hi_t = lax.dot_general(eye, hi, dn,
                           preferred_element_type=jnp.float32)
    xt = jnp.concatenate([lo_t, hi_t], axis=0)
    o_ref[0] = jnp.where(v[None, :], xt, bk_ref[...])


def _finish(gath, valid, backend_feature, eye):
    vr = valid.reshape(B, 1, HW)
    bk2 = backend_feature.reshape(C, 1)
    out = pl.pallas_call(
        _finish_body,
        grid=(B,),
        in_specs=[
            pl.BlockSpec((C // 2, C // 2), lambda b: (0, 0)),
            pl.BlockSpec((1, HW, C // 2), lambda b: (b, 0, 0)),
            pl.BlockSpec((1, 1, HW), lambda b: (b, 0, 0)),
            pl.BlockSpec((C, 1), lambda b: (0, 0)),
        ],
        out_specs=pl.BlockSpec((1, C, HW), lambda b: (b, 0, 0)),
        out_shape=jax.ShapeDtypeStruct((B, C, HW), jnp.float32),
    )(eye, gath, vr, bk2)
    return out.reshape(B, C, MAX_H, MAX_W)


def kernel(features, ys, xs, validation, backend_feature):
    feats = features.astype(jnp.float32)
    ysi = ys.astype(jnp.int32)
    xsi = xs.astype(jnp.int32)
    featT, rowmask = _transpose_feats(feats)
    tab = featT.reshape(B * P, C // 2)
    valid, gath = _sc_call(ysi, xsi, tab, rowmask.reshape(B, P))
    eye = jnp.eye(C // 2, dtype=jnp.bfloat16)
    return _finish(gath, valid, backend_feature.astype(jnp.float32), eye)
